# Initial kernel scaffold; baseline (speedup 1.0000x reference)
import jax
import jax.numpy as jnp
from jax.experimental import pallas as pl


def kernel(Yhat, Y):
    raise NotImplementedError("write your pallas kernel here")


# speedup vs baseline: 13.2446x; 13.2446x over previous
import functools

import jax
import jax.numpy as jnp
from jax import lax
from jax.experimental import pallas as pl
from jax.experimental.pallas import tpu as pltpu
from jax.experimental.pallas import tpu_sc as plsc

N = 131072
NC, NS, L = 2, 16, 16
NW = NC * NS
CHUNK = N // NW
B = 1024
FLAT = L * B


def _sc_body(yhat_hbm, y_hbm, s_out, c_out, l1_out, yh_v, y_v, sbin_v,
             cbin_v, l1_v):
    wid = lax.axis_index("s") * NC + lax.axis_index("c")
    base = wid * CHUNK
    pltpu.sync_copy(yhat_hbm.at[pl.ds(base, CHUNK)], yh_v)
    pltpu.sync_copy(y_hbm.at[pl.ds(base, CHUNK)], y_v)

    zero16 = jnp.zeros((L,), jnp.float32)
    one16 = jnp.ones((L,), jnp.float32)

    def zbody(i, carry):
        sbin_v[pl.ds(i * L, L)] = zero16
        cbin_v[pl.ds(i * L, L)] = zero16
        return carry

    lax.fori_loop(0, FLAT // L, zbody, 0)

    rowoff = lax.iota(jnp.int32, L) * B

    def body(i, l1):
        yh = yh_v[pl.ds(i * L, L)]
        y = y_v[pl.ds(i * L, L)]
        idx = rowoff + y
        ev = jnp.where(y > 0, one16, zero16)
        plsc.addupdate_scatter(sbin_v, [idx], jnp.exp(yh))
        plsc.addupdate_scatter(cbin_v, [idx], ev)
        return l1 + yh * ev

    l1 = lax.fori_loop(0, CHUNK // L, body, zero16)
    l1_v[...] = l1

    pltpu.sync_copy(sbin_v, s_out.at[wid])
    pltpu.sync_copy(cbin_v, c_out.at[wid])
    pltpu.sync_copy(l1_v, l1_out.at[wid])


_sc_call = functools.partial(
    pl.kernel,
    out_type=[
        jax.ShapeDtypeStruct((NW, FLAT), jnp.float32),
        jax.ShapeDtypeStruct((NW, FLAT), jnp.float32),
        jax.ShapeDtypeStruct((NW, L), jnp.float32),
    ],
    mesh=plsc.VectorSubcoreMesh(
        core_axis_name="c", subcore_axis_name="s", num_cores=NC,
        num_subcores=NS),
    scratch_types=[
        pltpu.VMEM((CHUNK,), jnp.float32),
        pltpu.VMEM((CHUNK,), jnp.int32),
        pltpu.VMEM((FLAT,), jnp.float32),
        pltpu.VMEM((FLAT,), jnp.float32),
        pltpu.VMEM((L,), jnp.float32),
    ],
    compiler_params=pltpu.CompilerParams(needs_layout_passes=False),
)(_sc_body)


_ROWS = NW * L
_STEPS = 32
_RPS = _ROWS // _STEPS


def _tc_body(s_ref, c_ref, l1_ref, out_ref, acc_s, acc_c):
    i = pl.program_id(0)
    ps = jnp.sum(s_ref[...], axis=0)
    pc = jnp.sum(c_ref[...], axis=0)

    @pl.when(i == 0)
    def _():
        acc_s[...] = ps
        acc_c[...] = pc

    @pl.when(i > 0)
    def _():
        acc_s[...] += ps
        acc_c[...] += pc

    @pl.when(i == _STEPS - 1)
    def _():
        s8 = acc_s[...]
        c8 = acc_c[...]
        li = lax.broadcasted_iota(jnp.int32, (128, 128), 0)
        lj = lax.broadcasted_iota(jnp.int32, (128, 128), 1)
        tri = (li <= lj).astype(jnp.float32)
        lanecum = jnp.dot(s8, tri, preferred_element_type=jnp.float32)
        totb = jnp.dot(s8, (li == li).astype(jnp.float32),
                       preferred_element_type=jnp.float32)
        ri = lax.broadcasted_iota(jnp.int32, (8, 8), 0)
        rj = lax.broadcasted_iota(jnp.int32, (8, 8), 1)
        stri = (rj < ri).astype(jnp.float32)
        rowcum = jnp.dot(stri, totb, preferred_element_type=jnp.float32)
        s_cum = lanecum + rowcum
        pos = c8 > 0.0
        loss2 = jnp.sum(jnp.where(pos, c8 * jnp.log(jnp.where(pos, s_cum, 1.0)),
                                  0.0))
        loss1 = jnp.sum(l1_ref[...])
        obs = jnp.sum(c8)
        out_ref[...] = jnp.zeros((8, 128), jnp.float32) + (loss2 - loss1) / obs


_tc_call = pl.pallas_call(
    _tc_body,
    grid=(_STEPS,),
    in_specs=[
        pl.BlockSpec((_RPS, 8, 128), lambda i: (i, 0, 0)),
        pl.BlockSpec((_RPS, 8, 128), lambda i: (i, 0, 0)),
        pl.BlockSpec((NW, L), lambda i: (0, 0)),
    ],
    out_specs=pl.BlockSpec((8, 128), lambda i: (0, 0)),
    out_shape=jax.ShapeDtypeStruct((8, 128), jnp.float32),
    scratch_shapes=[
        pltpu.VMEM((8, 128), jnp.float32),
        pltpu.VMEM((8, 128), jnp.float32),
    ],
)


def kernel(Yhat, Y):
    Yhat = jnp.squeeze(Yhat)
    Y = jnp.squeeze(Y)
    s_part, c_part, l1_part = _sc_call(Yhat, Y)
    out = _tc_call(s_part.reshape(_ROWS, 8, 128), c_part.reshape(_ROWS, 8, 128),
                   l1_part)
    return out[0, 0]


# speedup vs baseline: 19.8761x; 1.5007x over previous
import functools

import jax
import jax.numpy as jnp
from jax import lax
from jax.experimental import pallas as pl
from jax.experimental.pallas import tpu as pltpu
from jax.experimental.pallas import tpu_sc as plsc

N = 131072
NC, NS, L = 2, 16, 16
NW = NC * NS
CHUNK = N // NW
B = 1024
FLAT = L * B


def _sc_body(yhat_hbm, y_hbm, s_out, c_out, l1_out, yh_v, y_v, sbin_v,
             cbin_v, sred_v, cred_v, l1_v):
    wid = lax.axis_index("s") * NC + lax.axis_index("c")
    base = wid * CHUNK
    pltpu.sync_copy(yhat_hbm.at[pl.ds(base, CHUNK)], yh_v)
    pltpu.sync_copy(y_hbm.at[pl.ds(base, CHUNK)], y_v)

    zero16 = jnp.zeros((L,), jnp.float32)
    one16 = jnp.ones((L,), jnp.float32)

    def zbody(i, carry):
        sbin_v[pl.ds(i * L, L)] = zero16
        cbin_v[pl.ds(i * L, L)] = zero16
        return carry

    lax.fori_loop(0, FLAT // L, zbody, 0)

    rowoff = lax.iota(jnp.int32, L) * B

    def body(i, l1):
        yh = yh_v[pl.ds(i * L, L)]
        y = y_v[pl.ds(i * L, L)]
        idx = rowoff + y
        ev = jnp.where(y > 0, one16, zero16)
        plsc.addupdate_scatter(sbin_v, [idx], jnp.exp(yh))
        plsc.addupdate_scatter(cbin_v, [idx], ev)
        return l1 + yh * ev

    l1 = lax.fori_loop(0, CHUNK // L, body, zero16)
    l1_v[...] = l1

    def rbody(cidx, carry):
        def rsum(r, accs):
            a, b = accs
            a = a + sbin_v[pl.ds(r * B + cidx * L, L)]
            b = b + cbin_v[pl.ds(r * B + cidx * L, L)]
            return (a, b)

        acc_s, acc_c = lax.fori_loop(0, L, rsum, (zero16, zero16))
        sred_v[pl.ds(cidx * L, L)] = acc_s
        cred_v[pl.ds(cidx * L, L)] = acc_c
        return carry

    lax.fori_loop(0, B // L, rbody, 0)

    pltpu.sync_copy(sred_v, s_out.at[wid])
    pltpu.sync_copy(cred_v, c_out.at[wid])
    pltpu.sync_copy(l1_v, l1_out.at[wid])


_sc_call = functools.partial(
    pl.kernel,
    out_type=[
        jax.ShapeDtypeStruct((NW, B), jnp.float32),
        jax.ShapeDtypeStruct((NW, B), jnp.float32),
        jax.ShapeDtypeStruct((NW, L), jnp.float32),
    ],
    mesh=plsc.VectorSubcoreMesh(
        core_axis_name="c", subcore_axis_name="s", num_cores=NC,
        num_subcores=NS),
    scratch_types=[
        pltpu.VMEM((CHUNK,), jnp.float32),
        pltpu.VMEM((CHUNK,), jnp.int32),
        pltpu.VMEM((FLAT,), jnp.float32),
        pltpu.VMEM((FLAT,), jnp.float32),
        pltpu.VMEM((B,), jnp.float32),
        pltpu.VMEM((B,), jnp.float32),
        pltpu.VMEM((L,), jnp.float32),
    ],
    compiler_params=pltpu.CompilerParams(needs_layout_passes=False),
)(_sc_body)


def _tc_body(s_ref, c_ref, l1_ref, out_ref):
    s8 = jnp.sum(s_ref[...], axis=0)
    c8 = jnp.sum(c_ref[...], axis=0)
    li = lax.broadcasted_iota(jnp.int32, (128, 128), 0)
    lj = lax.broadcasted_iota(jnp.int32, (128, 128), 1)
    tri = (li <= lj).astype(jnp.float32)
    lanecum = jnp.dot(s8, tri, preferred_element_type=jnp.float32)
    totb = jnp.dot(s8, (li == li).astype(jnp.float32),
                   preferred_element_type=jnp.float32)
    ri = lax.broadcasted_iota(jnp.int32, (8, 8), 0)
    rj = lax.broadcasted_iota(jnp.int32, (8, 8), 1)
    stri = (rj < ri).astype(jnp.float32)
    rowcum = jnp.dot(stri, totb, preferred_element_type=jnp.float32)
    s_cum = lanecum + rowcum
    pos = c8 > 0.0
    loss2 = jnp.sum(jnp.where(pos, c8 * jnp.log(jnp.where(pos, s_cum, 1.0)),
                              0.0))
    loss1 = jnp.sum(l1_ref[...])
    obs = jnp.sum(c8)
    out_ref[...] = jnp.zeros((8, 128), jnp.float32) + (loss2 - loss1) / obs


_tc_call = pl.pallas_call(
    _tc_body,
    out_shape=jax.ShapeDtypeStruct((8, 128), jnp.float32),
)


def kernel(Yhat, Y):
    Yhat = jnp.squeeze(Yhat)
    Y = jnp.squeeze(Y)
    s_part, c_part, l1_part = _sc_call(Yhat, Y)
    out = _tc_call(s_part.reshape(NW, 8, 128), c_part.reshape(NW, 8, 128),
                   l1_part)
    return out[0, 0]


# speedup vs baseline: 23.1217x; 1.1633x over previous
import functools

import jax
import jax.numpy as jnp
from jax import lax
from jax.experimental import pallas as pl
from jax.experimental.pallas import tpu as pltpu
from jax.experimental.pallas import tpu_sc as plsc

N = 131072
NC, NS, L = 2, 16, 16
NW = NC * NS
CHUNK = N // NW
B = 1024
FLAT = L * B


def _sc_body(yhat_hbm, y_hbm, s_out, c_out, l1_out, yh_v, y_v, sbin_v,
             cbin_v, l1_v):
    wid = lax.axis_index("s") * NC + lax.axis_index("c")
    base = wid * CHUNK
    pltpu.sync_copy(yhat_hbm.at[pl.ds(base, CHUNK)], yh_v)
    pltpu.sync_copy(y_hbm.at[pl.ds(base, CHUNK)], y_v)

    zero16 = jnp.zeros((L,), jnp.float32)
    one16 = jnp.ones((L,), jnp.float32)

    def zbody(i, carry):
        sbin_v[pl.ds(i * L, L)] = zero16
        cbin_v[pl.ds(i * L, L)] = zero16
        return carry

    lax.fori_loop(0, B // L, zbody, 0)

    def body(i, l1):
        yh = yh_v[pl.ds(i * L, L)]
        y = y_v[pl.ds(i * L, L)]
        ev = jnp.where(y > 0, one16, zero16)
        plsc.addupdate_scatter(sbin_v, [y], jnp.exp(yh))
        plsc.addupdate_scatter(cbin_v, [y], ev)
        return l1 + yh * ev

    l1 = lax.fori_loop(0, CHUNK // L, body, zero16)
    l1_v[...] = l1

    pltpu.sync_copy(sbin_v, s_out.at[wid])
    pltpu.sync_copy(cbin_v, c_out.at[wid])
    pltpu.sync_copy(l1_v, l1_out.at[wid])


_sc_call = functools.partial(
    pl.kernel,
    out_type=[
        jax.ShapeDtypeStruct((NW, B), jnp.float32),
        jax.ShapeDtypeStruct((NW, B), jnp.float32),
        jax.ShapeDtypeStruct((NW, L), jnp.float32),
    ],
    mesh=plsc.VectorSubcoreMesh(
        core_axis_name="c", subcore_axis_name="s", num_cores=NC,
        num_subcores=NS),
    scratch_types=[
        pltpu.VMEM((CHUNK,), jnp.float32),
        pltpu.VMEM((CHUNK,), jnp.int32),
        pltpu.VMEM((B,), jnp.float32),
        pltpu.VMEM((B,), jnp.float32),
        pltpu.VMEM((L,), jnp.float32),
    ],
    compiler_params=pltpu.CompilerParams(needs_layout_passes=False),
)(_sc_body)


def _tc_body(s_ref, c_ref, l1_ref, out_ref):
    s8 = jnp.sum(s_ref[...], axis=0)
    c8 = jnp.sum(c_ref[...], axis=0)
    li = lax.broadcasted_iota(jnp.int32, (128, 128), 0)
    lj = lax.broadcasted_iota(jnp.int32, (128, 128), 1)
    tri = (li <= lj).astype(jnp.float32)
    lanecum = jnp.dot(s8, tri, preferred_element_type=jnp.float32)
    totb = jnp.dot(s8, (li == li).astype(jnp.float32),
                   preferred_element_type=jnp.float32)
    ri = lax.broadcasted_iota(jnp.int32, (8, 8), 0)
    rj = lax.broadcasted_iota(jnp.int32, (8, 8), 1)
    stri = (rj < ri).astype(jnp.float32)
    rowcum = jnp.dot(stri, totb, preferred_element_type=jnp.float32)
    s_cum = lanecum + rowcum
    pos = c8 > 0.0
    loss2 = jnp.sum(jnp.where(pos, c8 * jnp.log(jnp.where(pos, s_cum, 1.0)),
                              0.0))
    loss1 = jnp.sum(l1_ref[...])
    obs = jnp.sum(c8)
    out_ref[...] = jnp.zeros((8, 128), jnp.float32) + (loss2 - loss1) / obs


_tc_call = pl.pallas_call(
    _tc_body,
    out_shape=jax.ShapeDtypeStruct((8, 128), jnp.float32),
)


def kernel(Yhat, Y):
    Yhat = jnp.squeeze(Yhat)
    Y = jnp.squeeze(Y)
    s_part, c_part, l1_part = _sc_call(Yhat, Y)
    out = _tc_call(s_part.reshape(NW, 8, 128), c_part.reshape(NW, 8, 128),
                   l1_part)
    return out[0, 0]


# speedup vs baseline: 26.1219x; 1.1298x over previous
import functools

import jax
import jax.numpy as jnp
from jax import lax
from jax.experimental import pallas as pl
from jax.experimental.pallas import tpu as pltpu
from jax.experimental.pallas import tpu_sc as plsc

N = 131072
NC, NS, L = 2, 16, 16
NW = NC * NS
CHUNK = N // NW
B = 1024
FLAT = L * B


def _sc_body(yhat_hbm, y_hbm, s_out, c_out, l1_out, yh_v, y_v, sbin_v,
             cbin_v, l1_v):
    wid = lax.axis_index("s") * NC + lax.axis_index("c")
    base = wid * CHUNK
    pltpu.sync_copy(yhat_hbm.at[pl.ds(base, CHUNK)], yh_v)
    pltpu.sync_copy(y_hbm.at[pl.ds(base, CHUNK)], y_v)

    zero16 = jnp.zeros((L,), jnp.float32)
    one16 = jnp.ones((L,), jnp.float32)

    def zbody(i, carry):
        sbin_v[pl.ds(i * L, L)] = zero16
        cbin_v[pl.ds(i * L, L)] = zero16
        return carry

    lax.fori_loop(0, B // L, zbody, 0)

    def body(i, l1):
        yh = yh_v[pl.ds(i * L, L)]
        y = y_v[pl.ds(i * L, L)]
        ev = jnp.where(y > 0, one16, zero16)
        plsc.addupdate_scatter(sbin_v, [y], jnp.exp(yh))
        plsc.addupdate_scatter(cbin_v, [y], ev)
        return l1 + yh * ev

    l1 = lax.fori_loop(0, CHUNK // L, body, zero16)
    l1_v[...] = l1

    pltpu.sync_copy(sbin_v, s_out.at[wid])
    pltpu.sync_copy(cbin_v, c_out.at[wid])
    pltpu.sync_copy(l1_v, l1_out.at[wid])


_sc_call = functools.partial(
    pl.kernel,
    out_type=[
        jax.ShapeDtypeStruct((NW, B), jnp.float32),
        jax.ShapeDtypeStruct((NW, B), jnp.float32),
        jax.ShapeDtypeStruct((NW, L), jnp.float32),
    ],
    mesh=plsc.VectorSubcoreMesh(
        core_axis_name="c", subcore_axis_name="s", num_cores=NC,
        num_subcores=NS),
    scratch_types=[
        pltpu.VMEM((CHUNK,), jnp.float32),
        pltpu.VMEM((CHUNK,), jnp.int32),
        pltpu.VMEM((B,), jnp.float32),
        pltpu.VMEM((B,), jnp.float32),
        pltpu.VMEM((L,), jnp.float32),
    ],
    compiler_params=pltpu.CompilerParams(needs_layout_passes=False),
)(_sc_body)


def _tc_body(s_ref, c_ref, l1_ref, out_ref):
    s8 = jnp.sum(s_ref[...], axis=0).reshape(8, 128)
    c8 = jnp.sum(c_ref[...], axis=0).reshape(8, 128)
    li = lax.broadcasted_iota(jnp.int32, (128, 128), 0)
    lj = lax.broadcasted_iota(jnp.int32, (128, 128), 1)
    tri = (li <= lj).astype(jnp.float32)
    lanecum = jnp.dot(s8, tri, preferred_element_type=jnp.float32)
    totb = jnp.dot(s8, (li == li).astype(jnp.float32),
                   preferred_element_type=jnp.float32)
    ri = lax.broadcasted_iota(jnp.int32, (8, 8), 0)
    rj = lax.broadcasted_iota(jnp.int32, (8, 8), 1)
    stri = (rj < ri).astype(jnp.float32)
    rowcum = jnp.dot(stri, totb, preferred_element_type=jnp.float32)
    s_cum = lanecum + rowcum
    pos = c8 > 0.0
    loss2 = jnp.sum(jnp.where(pos, c8 * jnp.log(jnp.where(pos, s_cum, 1.0)),
                              0.0))
    loss1 = jnp.sum(l1_ref[...])
    obs = jnp.sum(c8)
    out_ref[0, 0] = (loss2 - loss1) / obs


_tc_call = pl.pallas_call(
    _tc_body,
    out_specs=pl.BlockSpec(memory_space=pltpu.MemorySpace.SMEM),
    out_shape=jax.ShapeDtypeStruct((1, 1), jnp.float32),
)


def kernel(Yhat, Y):
    Yhat = jnp.squeeze(Yhat)
    Y = jnp.squeeze(Y)
    s_part, c_part, l1_part = _sc_call(Yhat, Y)
    out = _tc_call(s_part, c_part, l1_part)
    return out[0, 0]


# speedup vs baseline: 27.1013x; 1.0375x over previous
import functools

import jax
import jax.numpy as jnp
from jax import lax
from jax.experimental import pallas as pl
from jax.experimental.pallas import tpu as pltpu
from jax.experimental.pallas import tpu_sc as plsc

N = 131072
NC, NS, L = 2, 16, 16
NW = NC * NS
CHUNK = N // NW
B = 1024
W = 2 * B + L


def _sc_body(yhat_hbm, y_hbm, out_hbm, yh_v, y_v, bins_v, sem1, sem2):
    wid = lax.axis_index("s") * NC + lax.axis_index("c")
    base = wid * CHUNK
    cp1 = pltpu.async_copy(yhat_hbm.at[pl.ds(base, CHUNK)], yh_v, sem1)
    cp2 = pltpu.async_copy(y_hbm.at[pl.ds(base, CHUNK)], y_v, sem2)

    zero16 = jnp.zeros((L,), jnp.float32)
    one16 = jnp.ones((L,), jnp.float32)

    def zbody(i, carry):
        bins_v[pl.ds(i * L, L)] = zero16
        return carry

    lax.fori_loop(0, 2 * B // L, zbody, 0)
    cp1.wait()
    cp2.wait()

    coff = jnp.full((L,), B, jnp.int32)

    def body(i, l1):
        for k in range(2):
            j = (2 * i + k) * L
            yh = yh_v[pl.ds(j, L)]
            y = y_v[pl.ds(j, L)]
            plsc.addupdate_scatter(bins_v, [y], jnp.exp(yh))
            plsc.addupdate_scatter(bins_v, [y + coff], one16)
            l1 = l1 + jnp.where(y > 0, yh, zero16)
        return l1

    l1 = lax.fori_loop(0, CHUNK // (2 * L), body, zero16)
    bins_v[pl.ds(2 * B, L)] = l1

    pltpu.sync_copy(bins_v, out_hbm.at[wid])


_sc_call = functools.partial(
    pl.kernel,
    out_type=[jax.ShapeDtypeStruct((NW, W), jnp.float32)],
    mesh=plsc.VectorSubcoreMesh(
        core_axis_name="c", subcore_axis_name="s", num_cores=NC,
        num_subcores=NS),
    scratch_types=[
        pltpu.VMEM((CHUNK,), jnp.float32),
        pltpu.VMEM((CHUNK,), jnp.int32),
        pltpu.VMEM((W,), jnp.float32),
        pltpu.SemaphoreType.DMA,
        pltpu.SemaphoreType.DMA,
    ],
    compiler_params=pltpu.CompilerParams(needs_layout_passes=False),
)(_sc_body)


def _tc_body(p_ref, out_ref):
    s8 = jnp.sum(p_ref[:, :B], axis=0).reshape(8, 128)
    c8 = jnp.sum(p_ref[:, B:2 * B], axis=0).reshape(8, 128)
    v0 = (lax.broadcasted_iota(jnp.int32, (8, 128), 0) +
          lax.broadcasted_iota(jnp.int32, (8, 128), 1)) > 0
    c8 = jnp.where(v0, c8, 0.0)
    li = lax.broadcasted_iota(jnp.int32, (128, 128), 0)
    lj = lax.broadcasted_iota(jnp.int32, (128, 128), 1)
    tri = (li <= lj).astype(jnp.float32)
    lanecum = jnp.dot(s8, tri, preferred_element_type=jnp.float32)
    totb = jnp.dot(s8, (li == li).astype(jnp.float32),
                   preferred_element_type=jnp.float32)
    ri = lax.broadcasted_iota(jnp.int32, (8, 8), 0)
    rj = lax.broadcasted_iota(jnp.int32, (8, 8), 1)
    stri = (rj < ri).astype(jnp.float32)
    rowcum = jnp.dot(stri, totb, preferred_element_type=jnp.float32)
    s_cum = lanecum + rowcum
    pos = c8 > 0.0
    loss2 = jnp.sum(jnp.where(pos, c8 * jnp.log(jnp.where(pos, s_cum, 1.0)),
                              0.0))
    loss1 = jnp.sum(p_ref[:, 2 * B:])
    obs = jnp.sum(c8)
    out_ref[0, 0] = (loss2 - loss1) / obs


_tc_call = pl.pallas_call(
    _tc_body,
    out_specs=pl.BlockSpec(memory_space=pltpu.MemorySpace.SMEM),
    out_shape=jax.ShapeDtypeStruct((1, 1), jnp.float32),
)


def kernel(Yhat, Y):
    Yhat = jnp.squeeze(Yhat)
    Y = jnp.squeeze(Y)
    (part,) = _sc_call(Yhat, Y)
    out = _tc_call(part)
    return out[0, 0]
